# baseline (speedup 1.0000x reference)
import functools

import jax
import jax.numpy as jnp
from jax import lax
from jax.experimental import pallas as pl
from jax.experimental.pallas import tpu as pltpu
from jax.experimental.pallas import tpu_sc as plsc

EPS = 1e-5
NEG_SLOPE = 0.2

_KA = 1000
_KM = 1000
_NP = 50176
_E2 = 864000

_SC_PARAMS = pltpu.CompilerParams(needs_layout_passes=False,
                                  use_tc_tiling_on_sc=False)

_PACK_EVEN_IS_A = True

_I2L = [(i % 2) * 16 + i // 2 for i in range(32)]
if _PACK_EVEN_IS_A:
    _ST2INT = [16 * (q % 2) + q // 2 for q in range(32)]
else:
    _ST2INT = [q // 2 + 16 * (1 - q % 2) for q in range(32)]
_QHALF = [_I2L[_ST2INT[q]] for q in range(32)]
_QPERM = _QHALF + [32 + q for q in _QHALF]
_INVQ = [0] * 64
for _q, _l in enumerate(_QPERM):
    _INVQ[_l] = _q


def _att_sc(src1, dst1, asrc, adst, ae_flat, M16):
    NP = asrc.shape[0]
    E = src1.shape[0]
    chunks = E // (32 * _KA)
    rows_per_tile = NP // 16
    f32 = jnp.float32

    mesh = plsc.VectorSubcoreMesh(core_axis_name="c", subcore_axis_name="s")

    @functools.partial(
        pl.kernel,
        out_type=(jax.ShapeDtypeStruct((E, 8), f32),
                  jax.ShapeDtypeStruct((2, NP, 8), f32)),
        mesh=mesh,
        compiler_params=_SC_PARAMS,
        scratch_types=dict(
            Dsh=pltpu.VMEM_SHARED((NP, 8), f32),
            srcv=pltpu.VMEM((_KA,), jnp.int32),
            dstv=pltpu.VMEM((_KA,), jnp.int32),
            asrcv=pltpu.VMEM((_KA, 8), f32),
            adstv=pltpu.VMEM((_KA, 8), f32),
            aev=pltpu.VMEM((_KA * 4,), f32),
            exv=pltpu.VMEM((_KA, 8), f32),
            Mv=pltpu.VMEM((16,), f32),
            sem1=pltpu.SemaphoreType.DMA,
            sem2=pltpu.SemaphoreType.DMA,
        ),
    )
    def k(src_h, dst_h, asrc_h, adst_h, ae_h, M_h, ex_out, den_out,
          Dsh, srcv, dstv, asrcv, adstv, aev, exv, Mv, sem1, sem2):
        c = lax.axis_index("c")
        s = lax.axis_index("s")
        z16 = jnp.zeros((16,), f32)
        iota = lax.iota(jnp.int32, 16)
        rb16 = lax.shift_right_logical(iota, 2)
        c16 = lax.bitwise_and(iota, 3)
        rb8 = lax.shift_right_logical(iota, 3)
        c8 = lax.bitwise_and(iota, 7)

        def zs(j, _):
            plsc.store_scatter(exv, [rb8 + 2 * j, c8], z16)
            return 0
        lax.fori_loop(0, _KA // 2, zs, 0)

        r0 = s * rows_per_tile
        pltpu.sync_copy(exv.at[pl.ds(0, _KA), :], Dsh.at[pl.ds(r0, _KA), :])
        pltpu.sync_copy(exv.at[pl.ds(0, _KA), :],
                        Dsh.at[pl.ds(r0 + _KA, _KA), :])
        pltpu.sync_copy(exv.at[pl.ds(0, _KA), :],
                        Dsh.at[pl.ds(r0 + 2 * _KA, _KA), :])
        pltpu.sync_copy(exv.at[pl.ds(0, 136), :],
                        Dsh.at[pl.ds(r0 + 3 * _KA, 136), :])
        plsc.subcore_barrier()

        pltpu.sync_copy(M_h, Mv)
        m16 = Mv[...]

        def chunk(i, _):
            base = ((c * 16 + s) * chunks + i) * _KA
            pltpu.sync_copy(src_h.at[pl.ds(base, _KA)], srcv)
            pltpu.sync_copy(dst_h.at[pl.ds(base, _KA)], dstv)
            pltpu.sync_copy(ae_h.at[pl.ds(base * 4, _KA * 4)], aev)
            cp1 = pltpu.async_copy(asrc_h.at[srcv], asrcv, sem1)
            cp2 = pltpu.async_copy(adst_h.at[dstv], adstv, sem2)
            cp1.wait()
            cp2.wait()

            def exbody(q, _):
                for u in range(2):
                    j = 2 * q + u
                    rowi = rb16 + 4 * j
                    a1 = plsc.load_gather(asrcv, [rowi, c16])
                    a2 = plsc.load_gather(adstv, [rowi, c16])
                    sv = a1 + a2 + aev[pl.ds(j * 16, 16)]
                    al = jnp.where(sv > 0, sv, NEG_SLOPE * sv)
                    plsc.store_scatter(exv, [rowi, c16], jnp.exp(al - m16))
                return 0
            lax.fori_loop(0, _KA * 4 // 32, exbody, 0)

            pltpu.sync_copy(exv, Dsh.at[dstv], add=True)
            pltpu.sync_copy(exv, ex_out.at[pl.ds(base, _KA), :])
            return 0

        lax.fori_loop(0, chunks, chunk, 0)
        plsc.subcore_barrier()

        pltpu.sync_copy(Dsh.at[pl.ds(r0, rows_per_tile), :],
                        den_out.at[c, pl.ds(r0, rows_per_tile), :])

    return k(src1, dst1, asrc, adst, ae_flat, M16)


def _msg_sc(src1, dst1, ex, xp2):
    NP = xp2.shape[1]
    E = src1.shape[0]
    chunks = E // (16 * _KM)
    rows_per_tile = NP // 16
    f32 = jnp.float32
    bf16 = jnp.bfloat16

    mesh = plsc.VectorSubcoreMesh(core_axis_name="c", subcore_axis_name="s")

    @functools.partial(
        pl.kernel,
        out_type=jax.ShapeDtypeStruct((2, NP, 32), bf16),
        mesh=mesh,
        compiler_params=_SC_PARAMS,
        scratch_types=dict(
            Ssh=pltpu.VMEM_SHARED((NP, 32), bf16),
            srcv=pltpu.VMEM((_KM,), jnp.int32),
            dstv=pltpu.VMEM((_KM,), jnp.int32),
            exv=pltpu.VMEM((_KM, 8), f32),
            xpv=pltpu.VMEM((_KM, 32), f32),
            xpbf=pltpu.VMEM((_KM, 32), bf16),
            sem1=pltpu.SemaphoreType.DMA,
        ),
    )
    def k(src_h, dst_h, ex_h, xp_h, S_out,
          Ssh, srcv, dstv, exv, xpv, xpbf, sem1):
        c = lax.axis_index("c")
        s = lax.axis_index("s")
        z32 = jnp.zeros((32,), bf16)
        iota = lax.iota(jnp.int32, 16)
        ialt = 2 * c + lax.bitwise_and(iota, 1)

        def zx(j, _):
            xpbf[j, :] = z32
            return 0
        lax.fori_loop(0, _KM, zx, 0)

        r0 = s * rows_per_tile
        pltpu.sync_copy(xpbf.at[pl.ds(0, _KM), :], Ssh.at[pl.ds(r0, _KM), :])
        pltpu.sync_copy(xpbf.at[pl.ds(0, _KM), :],
                        Ssh.at[pl.ds(r0 + _KM, _KM), :])
        pltpu.sync_copy(xpbf.at[pl.ds(0, _KM), :],
                        Ssh.at[pl.ds(r0 + 2 * _KM, _KM), :])
        pltpu.sync_copy(xpbf.at[pl.ds(0, 136), :],
                        Ssh.at[pl.ds(r0 + 3 * _KM, 136), :])
        plsc.subcore_barrier()

        def chunk(i, _):
            base = (s * chunks + i) * _KM
            pltpu.sync_copy(src_h.at[pl.ds(base, _KM)], srcv)
            pltpu.sync_copy(dst_h.at[pl.ds(base, _KM)], dstv)
            pltpu.sync_copy(ex_h.at[pl.ds(base, _KM), :], exv)
            cp1 = pltpu.async_copy(xp_h.at[c].at[srcv], xpv, sem1)
            cp1.wait()

            def msgbody(q, _):
                rbase = 8 * q
                for u in range(8):
                    r = rbase + u
                    rsp = jnp.full((16,), r, jnp.int32)
                    mult = plsc.load_gather(exv, [rsp, ialt])
                    m0 = xpv[r, pl.ds(0, 16)] * mult
                    m1 = xpv[r, pl.ds(16, 16)] * mult
                    xpbf[r, :] = plsc.pack(
                        m0, m1, format=plsc.PackFormat.INTERLEAVED)
                return 0
            lax.fori_loop(0, _KM // 8, msgbody, 0)

            pltpu.sync_copy(xpbf, Ssh.at[dstv], add=True)
            return 0

        lax.fori_loop(0, chunks, chunk, 0)
        plsc.subcore_barrier()

        pltpu.sync_copy(Ssh.at[pl.ds(r0, rows_per_tile), :],
                        S_out.at[c, pl.ds(r0, rows_per_tile), :])

    return k(src1, dst1, ex, xp2)


def _pool_sc(h, batch):
    N = h.shape[0]
    G = 1024
    f32 = jnp.float32
    KP = 400
    nchunk = N // KP
    mesh = plsc.VectorSubcoreMesh(core_axis_name="c", subcore_axis_name="s")

    @functools.partial(
        pl.kernel,
        out_type=jax.ShapeDtypeStruct((2, G, 64), f32),
        mesh=mesh,
        compiler_params=_SC_PARAMS,
        scratch_types=dict(
            Psh=pltpu.VMEM_SHARED((G, 64), f32),
            bv=pltpu.VMEM((KP,), jnp.int32),
            hv=pltpu.VMEM((KP, 64), f32),
        ),
    )
    def k(h_h, b_h, P_out, Psh, bv, hv):
        c = lax.axis_index("c")
        s = lax.axis_index("s")
        w = c * 16 + s
        z16 = jnp.zeros((16,), f32)

        def zx(j, _):
            for u in range(4):
                hv[j, pl.ds(16 * u, 16)] = z16
            return 0
        lax.fori_loop(0, KP, zx, 0)

        pltpu.sync_copy(hv.at[pl.ds(0, 64), :], Psh.at[pl.ds(s * 64, 64), :])
        plsc.subcore_barrier()

        nb = 3 + jnp.where(w < nchunk - 3 * 32, 1, 0)

        def chunk(i, _):
            base = (i * 32 + w) * KP
            pltpu.sync_copy(b_h.at[pl.ds(base, KP)], bv)
            pltpu.sync_copy(h_h.at[pl.ds(base, KP), :], hv)
            pltpu.sync_copy(hv, Psh.at[bv], add=True)
            return 0

        lax.fori_loop(0, nb, chunk, 0)
        plsc.subcore_barrier()

        pltpu.sync_copy(Psh.at[pl.ds(s * 64, 64), :],
                        P_out.at[c, pl.ds(s * 64, 64), :])

    return k(h, batch)


_BLK = 2000


def _nodeA_body(SA_ref, SB_ref, den_ref, bias_ref, out_ref, stats_ref):
    SA = SA_ref[...].astype(jnp.float32)
    SB = SB_ref[...].astype(jnp.float32)
    den = den_ref[...]
    BLK = SA.shape[0]
    inv = 1.0 / (den + 1e-16)

    def expand2(v):
        return jnp.broadcast_to(v[:, None, :, None],
                                (BLK, 8, 2, 2)).reshape(BLK, 32)

    out = (jnp.concatenate([SA * expand2(inv[:, :2]),
                            SB * expand2(inv[:, 2:])], axis=1)
           + bias_ref[...])
    out_ref[...] = out
    stats_ref[0, 0, :] = jnp.sum(out, axis=0)
    stats_ref[0, 1, :] = jnp.sum(out * out, axis=0)


def _nodeB_body(out_ref, scale_ref, shift_ref, W_ref, h_ref, xpA_ref, xpB_ref,
                as_ref, ad_ref):
    h = jnp.maximum(out_ref[...] * scale_ref[...] + shift_ref[...], 0.0)
    h_ref[...] = h
    P = jnp.dot(h, W_ref[...], preferred_element_type=jnp.float32)
    xpA_ref[...] = P[:, :32]
    xpB_ref[...] = P[:, 32:64]
    as_ref[...] = P[:, 64:68]
    ad_ref[...] = P[:, 68:72]


def _node_update(SA, SB, den, bias_q, gamma_q, beta_q, W_q):
    N = den.shape[0]
    HIDDEN = 64
    nblk = N // _BLK
    f32 = jnp.float32
    out, stats = pl.pallas_call(
        _nodeA_body,
        grid=(nblk,),
        in_specs=[pl.BlockSpec((_BLK, 32), lambda i: (i, 0)),
                  pl.BlockSpec((_BLK, 32), lambda i: (i, 0)),
                  pl.BlockSpec((_BLK, 4), lambda i: (i, 0)),
                  pl.BlockSpec((1, HIDDEN), lambda i: (0, 0))],
        out_specs=(pl.BlockSpec((_BLK, HIDDEN), lambda i: (i, 0)),
                   pl.BlockSpec((1, 2, HIDDEN), lambda i: (i, 0, 0))),
        out_shape=(jax.ShapeDtypeStruct((N, HIDDEN), f32),
                   jax.ShapeDtypeStruct((nblk, 2, HIDDEN), f32)),
    )(SA, SB, den, bias_q.reshape(1, HIDDEN))
    tot = jnp.sum(stats, axis=0)
    mu = tot[0] / N
    var = tot[1] / N - mu * mu
    scale = gamma_q * lax.rsqrt(var + EPS)
    shift = beta_q - mu * scale
    h, nxpA, nxpB, nas, nad = pl.pallas_call(
        _nodeB_body,
        grid=(nblk,),
        in_specs=[pl.BlockSpec((_BLK, HIDDEN), lambda i: (i, 0)),
                  pl.BlockSpec((1, HIDDEN), lambda i: (0, 0)),
                  pl.BlockSpec((1, HIDDEN), lambda i: (0, 0)),
                  pl.BlockSpec((HIDDEN, 72), lambda i: (0, 0))],
        out_specs=(pl.BlockSpec((_BLK, HIDDEN), lambda i: (i, 0)),
                   pl.BlockSpec((_BLK, 32), lambda i: (i, 0)),
                   pl.BlockSpec((_BLK, 32), lambda i: (i, 0)),
                   pl.BlockSpec((_BLK, 4), lambda i: (i, 0)),
                   pl.BlockSpec((_BLK, 4), lambda i: (i, 0))),
        out_shape=(jax.ShapeDtypeStruct((N, HIDDEN), f32),
                   jax.ShapeDtypeStruct((N, 32), f32),
                   jax.ShapeDtypeStruct((N, 32), f32),
                   jax.ShapeDtypeStruct((N, 4), f32),
                   jax.ShapeDtypeStruct((N, 4), f32)),
    )(out, scale.reshape(1, HIDDEN), shift.reshape(1, HIDDEN), W_q)
    return h, nxpA, nxpB, nas, nad


def kernel(x, edge_index, edge_attr, batch, node_W, node_b, edge_W, edge_b,
           lin_W, lin_edge_W, att_src, att_dst, att_edge, gat_bias, bn_gamma,
           bn_beta):
    N, NODE_DIM = x.shape
    E = edge_index.shape[1]
    LAYERS, HIDDEN, HO = lin_W.shape
    HEADS, OUT = att_src.shape[1], att_src.shape[2]
    G = 1024
    f32 = jnp.float32
    NP = _NP
    PAD = _E2 - E - N

    qperm = jnp.array(_QPERM, jnp.int32)
    invq = jnp.array(_INVQ, jnp.int32)
    colA = jnp.array(_I2L, jnp.int32)
    colB = colA + 32

    loop_idx = jnp.arange(N, dtype=jnp.int32)
    pad_idx = jnp.full((PAD,), N, jnp.int32)
    src2 = jnp.concatenate([edge_index[0], loop_idx, pad_idx])
    dst2 = jnp.concatenate([edge_index[1], loop_idx, pad_idx])

    lw = lin_W.reshape(LAYERS, HIDDEN, HEADS, OUT)
    lew = lin_edge_W.reshape(LAYERS, HIDDEN, HEADS, OUT)
    U_src = jnp.einsum('lkho,lho->lkh', lw, att_src)
    U_dst = jnp.einsum('lkho,lho->lkh', lw, att_dst)
    V_e = jnp.einsum('lkho,lho->lkh', lew, att_edge)

    P_e = jnp.einsum('dk,lkh->ldh', edge_W, V_e)
    q_e = jnp.einsum('k,lkh->lh', edge_b, V_e)
    ae = jnp.einsum('ed,ldh->leh', edge_attr, P_e) + q_e[:, None, :]
    ea_mean = edge_attr.mean(axis=0) @ edge_W + edge_b
    a_loop = jnp.einsum('k,lkh->lh', ea_mean, V_e)
    ae_max = jnp.max(ae, axis=1)

    ae2 = jnp.concatenate([
        ae,
        jnp.broadcast_to(a_loop[:, None, :], (LAYERS, N, HEADS)),
        jnp.full((LAYERS, PAD, HEADS), -1e30, f32),
    ], axis=1)

    h = x @ node_W + node_b

    W0 = jnp.concatenate([lin_W[0][:, colA], lin_W[0][:, colB],
                          U_src[0], U_dst[0]], axis=1)
    P0 = h @ W0
    xpA, xpB = P0[:, :32], P0[:, 32:64]
    a_s, a_d = P0[:, 64:64 + HEADS], P0[:, 64 + HEADS:64 + 2 * HEADS]

    for l in range(LAYERS):
        bound = (jnp.max(a_s, axis=0) + jnp.max(a_d, axis=0)
                 + jnp.maximum(ae_max[l], a_loop[l]))
        M = jnp.where(bound > 0, bound, NEG_SLOPE * bound)
        M16 = jnp.tile(M, 4)

        xp2 = (jnp.zeros((2, NP, 32), f32)
               .at[0, :N].set(xpA).at[1, :N].set(xpB))
        a_sp = jnp.zeros((NP, 8), f32).at[:N, :4].set(a_s)
        a_dp = jnp.zeros((NP, 8), f32).at[:N, :4].set(a_d)
        ex_e, den2 = _att_sc(src2, dst2, a_sp, a_dp, ae2[l].reshape(-1), M16)
        S2 = _msg_sc(src2, dst2, ex_e, xp2)
        den_tot = (den2[0] + den2[1])[:N, :4]

        if l + 1 < LAYERS:
            W_next = jnp.concatenate(
                [lin_W[l + 1][:, colA], lin_W[l + 1][:, colB],
                 U_src[l + 1], U_dst[l + 1]], axis=1)[qperm]
        else:
            W_next = jnp.zeros((HIDDEN, HIDDEN + 2 * HEADS), f32)
        h, xpA, xpB, a_s, a_d = _node_update(
            S2[0, :N], S2[1, :N], den_tot, gat_bias[l][qperm],
            bn_gamma[l][qperm], bn_beta[l][qperm], W_next)

    P2 = _pool_sc(h, batch)
    sums = (P2[0] + P2[1])[:, invq]
    bounds = jnp.searchsorted(batch, jnp.arange(G + 1, dtype=jnp.int32))
    counts = (bounds[1:] - bounds[:-1]).astype(f32)
    return sums / jnp.maximum(counts, 1.0)[:, None]


# speedup vs baseline: 1.1499x; 1.1499x over previous
import functools

import jax
import jax.numpy as jnp
from jax import lax
from jax.experimental import pallas as pl
from jax.experimental.pallas import tpu as pltpu
from jax.experimental.pallas import tpu_sc as plsc

EPS = 1e-5
NEG_SLOPE = 0.2

_KA = 1000
_KM = 400

_SC_PARAMS = pltpu.CompilerParams(needs_layout_passes=False,
                                  use_tc_tiling_on_sc=False)


def _att_sc(src1, dst1, asrc, adst, ae_flat, M16):
    NP = asrc.shape[0]
    E = src1.shape[0]
    chunks = E // (32 * _KA)
    rows_per_tile = NP // 16
    f32 = jnp.float32

    mesh = plsc.VectorSubcoreMesh(core_axis_name="c", subcore_axis_name="s")

    @functools.partial(
        pl.kernel,
        out_type=(jax.ShapeDtypeStruct((E, 8), f32),
                  jax.ShapeDtypeStruct((2, NP, 8), f32)),
        mesh=mesh,
        compiler_params=_SC_PARAMS,
        scratch_types=dict(
            Dsh=pltpu.VMEM_SHARED((NP, 8), f32),
            srcv=pltpu.VMEM((_KA,), jnp.int32),
            dstv=pltpu.VMEM((_KA,), jnp.int32),
            asrcv=pltpu.VMEM((_KA, 8), f32),
            adstv=pltpu.VMEM((_KA, 8), f32),
            aev=pltpu.VMEM((_KA * 4,), f32),
            exv=pltpu.VMEM((_KA, 8), f32),
            Mv=pltpu.VMEM((16,), f32),
            sem1=pltpu.SemaphoreType.DMA,
            sem2=pltpu.SemaphoreType.DMA,
        ),
    )
    def k(src_h, dst_h, asrc_h, adst_h, ae_h, M_h, ex_out, den_out,
          Dsh, srcv, dstv, asrcv, adstv, aev, exv, Mv, sem1, sem2):
        c = lax.axis_index("c")
        s = lax.axis_index("s")
        z16 = jnp.zeros((16,), f32)
        iota = lax.iota(jnp.int32, 16)
        rb16 = lax.shift_right_logical(iota, 2)
        c16 = lax.bitwise_and(iota, 3)
        rb8 = lax.shift_right_logical(iota, 3)
        c8 = lax.bitwise_and(iota, 7)

        def zs(j, _):
            plsc.store_scatter(exv, [rb8 + 2 * j, c8], z16)
            return 0
        lax.fori_loop(0, _KA // 2, zs, 0)

        r0 = s * rows_per_tile
        pltpu.sync_copy(exv.at[pl.ds(0, _KA), :], Dsh.at[pl.ds(r0, _KA), :])
        pltpu.sync_copy(exv.at[pl.ds(0, _KA), :],
                        Dsh.at[pl.ds(r0 + _KA, _KA), :])
        pltpu.sync_copy(exv.at[pl.ds(0, _KA), :],
                        Dsh.at[pl.ds(r0 + 2 * _KA, _KA), :])
        pltpu.sync_copy(exv.at[pl.ds(0, 136), :],
                        Dsh.at[pl.ds(r0 + 3 * _KA, 136), :])
        plsc.subcore_barrier()

        pltpu.sync_copy(M_h, Mv)
        m16 = Mv[...]

        def chunk(i, _):
            base = ((c * 16 + s) * chunks + i) * _KA
            pltpu.sync_copy(src_h.at[pl.ds(base, _KA)], srcv)
            pltpu.sync_copy(dst_h.at[pl.ds(base, _KA)], dstv)
            pltpu.sync_copy(ae_h.at[pl.ds(base * 4, _KA * 4)], aev)
            cp1 = pltpu.async_copy(asrc_h.at[srcv], asrcv, sem1)
            cp2 = pltpu.async_copy(adst_h.at[dstv], adstv, sem2)
            cp1.wait()
            cp2.wait()

            def exbody(q, _):
                for u in range(2):
                    j = 2 * q + u
                    rowi = rb16 + 4 * j
                    a1 = plsc.load_gather(asrcv, [rowi, c16])
                    a2 = plsc.load_gather(adstv, [rowi, c16])
                    sv = a1 + a2 + aev[pl.ds(j * 16, 16)]
                    al = jnp.where(sv > 0, sv, NEG_SLOPE * sv)
                    plsc.store_scatter(exv, [rowi, c16], jnp.exp(al - m16))
                return 0
            lax.fori_loop(0, _KA * 4 // 32, exbody, 0)

            pltpu.sync_copy(exv, Dsh.at[dstv], add=True)
            pltpu.sync_copy(exv, ex_out.at[pl.ds(base, _KA), :])
            return 0

        lax.fori_loop(0, chunks, chunk, 0)
        plsc.subcore_barrier()

        pltpu.sync_copy(Dsh.at[pl.ds(r0, rows_per_tile), :],
                        den_out.at[c, pl.ds(r0, rows_per_tile), :])

    return k(src1, dst1, asrc, adst, ae_flat, M16)


def _msg_sc(src1, dst1, ex, xp2):
    NP = xp2.shape[1]
    E = src1.shape[0]
    chunks = E // (16 * _KM)
    rows_per_tile = NP // 16
    f32 = jnp.float32

    mesh = plsc.VectorSubcoreMesh(core_axis_name="c", subcore_axis_name="s")

    @functools.partial(
        pl.kernel,
        out_type=jax.ShapeDtypeStruct((2, NP, 32), f32),
        mesh=mesh,
        compiler_params=_SC_PARAMS,
        scratch_types=dict(
            Ssh=pltpu.VMEM_SHARED((NP, 32), f32),
            srcv=pltpu.VMEM((_KM,), jnp.int32),
            dstv=pltpu.VMEM((_KM,), jnp.int32),
            exv=pltpu.VMEM((_KM, 8), f32),
            xpv=pltpu.VMEM((_KM, 32), f32),
            sem1=pltpu.SemaphoreType.DMA,
        ),
    )
    def k(src_h, dst_h, ex_h, xp_h, S_out,
          Ssh, srcv, dstv, exv, xpv, sem1):
        c = lax.axis_index("c")
        s = lax.axis_index("s")
        z16 = jnp.zeros((16,), f32)

        def zx(j, _):
            xpv[j, pl.ds(0, 16)] = z16
            xpv[j, pl.ds(16, 16)] = z16
            return 0
        lax.fori_loop(0, _KM, zx, 0)

        r0 = s * rows_per_tile
        for tpart in range(7):
            pltpu.sync_copy(xpv.at[pl.ds(0, _KM), :],
                            Ssh.at[pl.ds(r0 + tpart * _KM, _KM), :])
        pltpu.sync_copy(xpv.at[pl.ds(0, 336), :],
                        Ssh.at[pl.ds(r0 + 7 * _KM, 336), :])
        plsc.subcore_barrier()

        cs0 = jnp.full((16,), 2 * c, jnp.int32)
        cs1 = jnp.full((16,), 2 * c + 1, jnp.int32)

        def chunk(i, _):
            base = (s * chunks + i) * _KM
            pltpu.sync_copy(src_h.at[pl.ds(base, _KM)], srcv)
            pltpu.sync_copy(dst_h.at[pl.ds(base, _KM)], dstv)
            pltpu.sync_copy(ex_h.at[pl.ds(base, _KM), :], exv)
            cp1 = pltpu.async_copy(xp_h.at[c].at[srcv], xpv, sem1)
            cp1.wait()

            def msgbody(q, _):
                rbase = 8 * q
                for u in range(8):
                    r = rbase + u
                    rsp = jnp.full((16,), r, jnp.int32)
                    e0 = plsc.load_gather(exv, [rsp, cs0])
                    e1 = plsc.load_gather(exv, [rsp, cs1])
                    xpv[r, pl.ds(0, 16)] = xpv[r, pl.ds(0, 16)] * e0
                    xpv[r, pl.ds(16, 16)] = xpv[r, pl.ds(16, 16)] * e1
                return 0
            lax.fori_loop(0, _KM // 8, msgbody, 0)

            pltpu.sync_copy(xpv, Ssh.at[dstv], add=True)
            return 0

        lax.fori_loop(0, chunks, chunk, 0)
        plsc.subcore_barrier()

        pltpu.sync_copy(Ssh.at[pl.ds(r0, rows_per_tile), :],
                        S_out.at[c, pl.ds(r0, rows_per_tile), :])

    return k(src1, dst1, ex, xp2)


def _pool_sc(h, batch):
    N = h.shape[0]
    G = 1024
    f32 = jnp.float32
    KP = 400
    nchunk = N // KP
    mesh = plsc.VectorSubcoreMesh(core_axis_name="c", subcore_axis_name="s")

    @functools.partial(
        pl.kernel,
        out_type=jax.ShapeDtypeStruct((2, G, 64), f32),
        mesh=mesh,
        compiler_params=_SC_PARAMS,
        scratch_types=dict(
            Psh=pltpu.VMEM_SHARED((G, 64), f32),
            bv=pltpu.VMEM((KP,), jnp.int32),
            hv=pltpu.VMEM((KP, 64), f32),
        ),
    )
    def k(h_h, b_h, P_out, Psh, bv, hv):
        c = lax.axis_index("c")
        s = lax.axis_index("s")
        w = c * 16 + s
        z16 = jnp.zeros((16,), f32)

        def zx(j, _):
            for u in range(4):
                hv[j, pl.ds(16 * u, 16)] = z16
            return 0
        lax.fori_loop(0, KP, zx, 0)

        pltpu.sync_copy(hv.at[pl.ds(0, 64), :], Psh.at[pl.ds(s * 64, 64), :])
        plsc.subcore_barrier()

        nb = 3 + jnp.where(w < nchunk - 3 * 32, 1, 0)

        def chunk(i, _):
            base = (i * 32 + w) * KP
            pltpu.sync_copy(b_h.at[pl.ds(base, KP)], bv)
            pltpu.sync_copy(h_h.at[pl.ds(base, KP), :], hv)
            pltpu.sync_copy(hv, Psh.at[bv], add=True)
            return 0

        lax.fori_loop(0, nb, chunk, 0)
        plsc.subcore_barrier()

        pltpu.sync_copy(Psh.at[pl.ds(s * 64, 64), :],
                        P_out.at[c, pl.ds(s * 64, 64), :])

    return k(h, batch)


_BLK = 2000


def _nodeA_body(SA_ref, SB_ref, den_ref, xpA_ref, xpB_ref, exl_ref, bias_ref,
                out_ref, stats_ref):
    SA = SA_ref[...]
    SB = SB_ref[...]
    den = den_ref[...]
    exl = exl_ref[...]
    BLK = SA.shape[0]
    inv = 1.0 / (den + 1e-16)

    def expand2(v):
        return jnp.broadcast_to(v[:, :, None], (BLK, 2, 16)).reshape(BLK, 32)

    outA = (SA + xpA_ref[...] * expand2(exl[:, :2])) * expand2(inv[:, :2])
    outB = (SB + xpB_ref[...] * expand2(exl[:, 2:])) * expand2(inv[:, 2:])
    out = jnp.concatenate([outA, outB], axis=1) + bias_ref[...]
    out_ref[...] = out
    stats_ref[0, 0, :] = jnp.sum(out, axis=0)
    stats_ref[0, 1, :] = jnp.sum(out * out, axis=0)


def _nodeB_body(out_ref, scale_ref, shift_ref, W_ref, h_ref, xpA_ref, xpB_ref,
                as_ref, ad_ref):
    h = jnp.maximum(out_ref[...] * scale_ref[...] + shift_ref[...], 0.0)
    h_ref[...] = h
    P = jnp.dot(h, W_ref[...], preferred_element_type=jnp.float32)
    xpA_ref[...] = P[:, :32]
    xpB_ref[...] = P[:, 32:64]
    as_ref[...] = P[:, 64:68]
    ad_ref[...] = P[:, 68:72]


def _node_update(SA, SB, den, xpA, xpB, exl, bias, gamma, beta, W):
    N = SA.shape[0]
    HIDDEN = 64
    nblk = N // _BLK
    f32 = jnp.float32
    out, stats = pl.pallas_call(
        _nodeA_body,
        grid=(nblk,),
        in_specs=[pl.BlockSpec((_BLK, 32), lambda i: (i, 0)),
                  pl.BlockSpec((_BLK, 32), lambda i: (i, 0)),
                  pl.BlockSpec((_BLK, 4), lambda i: (i, 0)),
                  pl.BlockSpec((_BLK, 32), lambda i: (i, 0)),
                  pl.BlockSpec((_BLK, 32), lambda i: (i, 0)),
                  pl.BlockSpec((_BLK, 4), lambda i: (i, 0)),
                  pl.BlockSpec((1, HIDDEN), lambda i: (0, 0))],
        out_specs=(pl.BlockSpec((_BLK, HIDDEN), lambda i: (i, 0)),
                   pl.BlockSpec((1, 2, HIDDEN), lambda i: (i, 0, 0))),
        out_shape=(jax.ShapeDtypeStruct((N, HIDDEN), f32),
                   jax.ShapeDtypeStruct((nblk, 2, HIDDEN), f32)),
    )(SA, SB, den, xpA, xpB, exl, bias.reshape(1, HIDDEN))
    tot = jnp.sum(stats, axis=0)
    mu = tot[0] / N
    var = tot[1] / N - mu * mu
    scale = gamma * lax.rsqrt(var + EPS)
    shift = beta - mu * scale
    h, nxpA, nxpB, nas, nad = pl.pallas_call(
        _nodeB_body,
        grid=(nblk,),
        in_specs=[pl.BlockSpec((_BLK, HIDDEN), lambda i: (i, 0)),
                  pl.BlockSpec((1, HIDDEN), lambda i: (0, 0)),
                  pl.BlockSpec((1, HIDDEN), lambda i: (0, 0)),
                  pl.BlockSpec((HIDDEN, 72), lambda i: (0, 0))],
        out_specs=(pl.BlockSpec((_BLK, HIDDEN), lambda i: (i, 0)),
                   pl.BlockSpec((_BLK, 32), lambda i: (i, 0)),
                   pl.BlockSpec((_BLK, 32), lambda i: (i, 0)),
                   pl.BlockSpec((_BLK, 4), lambda i: (i, 0)),
                   pl.BlockSpec((_BLK, 4), lambda i: (i, 0))),
        out_shape=(jax.ShapeDtypeStruct((N, HIDDEN), f32),
                   jax.ShapeDtypeStruct((N, 32), f32),
                   jax.ShapeDtypeStruct((N, 32), f32),
                   jax.ShapeDtypeStruct((N, 4), f32),
                   jax.ShapeDtypeStruct((N, 4), f32)),
    )(out, scale.reshape(1, HIDDEN), shift.reshape(1, HIDDEN), W)
    return h, nxpA, nxpB, nas, nad


def kernel(x, edge_index, edge_attr, batch, node_W, node_b, edge_W, edge_b,
           lin_W, lin_edge_W, att_src, att_dst, att_edge, gat_bias, bn_gamma,
           bn_beta):
    N, NODE_DIM = x.shape
    NP = 50176
    E = edge_index.shape[1]
    LAYERS, HIDDEN, HO = lin_W.shape
    HEADS, OUT = att_src.shape[1], att_src.shape[2]
    G = 1024
    f32 = jnp.float32

    src1 = edge_index[0]
    dst1 = edge_index[1]

    lw = lin_W.reshape(LAYERS, HIDDEN, HEADS, OUT)
    lew = lin_edge_W.reshape(LAYERS, HIDDEN, HEADS, OUT)
    U_src = jnp.einsum('lkho,lho->lkh', lw, att_src)
    U_dst = jnp.einsum('lkho,lho->lkh', lw, att_dst)
    V_e = jnp.einsum('lkho,lho->lkh', lew, att_edge)

    P_e = jnp.einsum('dk,lkh->ldh', edge_W, V_e)
    q_e = jnp.einsum('k,lkh->lh', edge_b, V_e)
    ae = jnp.einsum('ed,ldh->leh', edge_attr, P_e) + q_e[:, None, :]
    ea_mean = edge_attr.mean(axis=0) @ edge_W + edge_b
    a_loop = jnp.einsum('k,lkh->lh', ea_mean, V_e)
    ae_max = jnp.max(ae, axis=1)

    h = x @ node_W + node_b

    P0 = h @ jnp.concatenate([lin_W[0], U_src[0], U_dst[0]], axis=1)
    xpA, xpB = P0[:, :32], P0[:, 32:64]
    a_s, a_d = P0[:, 64:64 + HEADS], P0[:, 64 + HEADS:64 + 2 * HEADS]

    for l in range(LAYERS):
        bound = (jnp.max(a_s, axis=0) + jnp.max(a_d, axis=0)
                 + jnp.maximum(ae_max[l], a_loop[l]))
        M = jnp.where(bound > 0, bound, NEG_SLOPE * bound)
        M16 = jnp.tile(M, 4)

        s_l = a_s + a_d + a_loop[l]
        alpha_l = jnp.where(s_l > 0, s_l, NEG_SLOPE * s_l)
        exl = jnp.exp(alpha_l - M)

        xp2 = (jnp.zeros((2, NP, 32), f32)
               .at[0, :N].set(xpA).at[1, :N].set(xpB))
        a_sp = jnp.zeros((NP, 8), f32).at[:N, :4].set(a_s)
        a_dp = jnp.zeros((NP, 8), f32).at[:N, :4].set(a_d)
        ex_e, den2 = _att_sc(src1, dst1, a_sp, a_dp, ae[l].reshape(-1), M16)
        S2 = _msg_sc(src1, dst1, ex_e, xp2)
        den_tot = (den2[0] + den2[1])[:N, :4] + exl

        if l + 1 < LAYERS:
            W_next = jnp.concatenate(
                [lin_W[l + 1], U_src[l + 1], U_dst[l + 1]], axis=1)
        else:
            W_next = jnp.zeros((HIDDEN, HIDDEN + 2 * HEADS), f32)
        h, xpA, xpB, a_s, a_d = _node_update(
            S2[0, :N], S2[1, :N], den_tot, xpA, xpB, exl, gat_bias[l],
            bn_gamma[l], bn_beta[l], W_next)

    P2 = _pool_sc(h, batch)
    sums = P2[0] + P2[1]
    bounds = jnp.searchsorted(batch, jnp.arange(G + 1, dtype=jnp.int32))
    counts = (bounds[1:] - bounds[:-1]).astype(f32)
    return sums / jnp.maximum(counts, 1.0)[:, None]
